# Initial kernel scaffold; baseline (speedup 1.0000x reference)
import jax
import jax.numpy as jnp
from jax.experimental import pallas as pl


def kernel(x, embeddings):
    raise NotImplementedError("write your pallas kernel here")


# speedup vs baseline: 1.2230x; 1.2230x over previous
import functools
import jax
import jax.numpy as jnp
from jax import lax
from jax.experimental import pallas as pl


def _tree_sum_rows(a):
    n = a.shape[0]
    while n > 1:
        n //= 2
        a = a[:n] + a[n:]
    return a


def _tree_sum_cols(a):
    n = a.shape[1]
    while n > 1:
        n //= 2
        a = a[:, :n] + a[:, n:]
    return a


def _vq_kernel(x_ref, xv_ref, emb_ref, q_ref, part_ref):
    xb = x_ref[0]
    e = emb_ref[...]
    xv = xv_ref[0]

    xsq = _tree_sum_rows(xb * xb)
    esq = _tree_sum_cols(e * e)

    mm = lax.dot_general(e, xb, (((1,), (0,)), ((), ())),
                         preferred_element_type=jnp.float32)
    dist = (xsq + esq) - 2.0 * mm

    n = e.shape[0]
    t = xb.shape[1]
    iota_c = lax.broadcasted_iota(jnp.int32, (n, t), 0)
    mn = jnp.min(dist, axis=0, keepdims=True)
    idx = jnp.min(jnp.where(dist == mn, iota_c, n), axis=0)

    onehot = (iota_c == idx[None, :]).astype(jnp.float32)
    q = lax.dot_general(onehot, e, (((0,), (0,)), ((), ())),
                        preferred_element_type=jnp.float32,
                        precision=lax.Precision.HIGHEST)
    q_ref[0] = q

    diff = q - xv
    part = jnp.sum(diff * diff)
    part_ref[0, 0] = jnp.broadcast_to(part, (128,))


def kernel(x, embeddings):
    B, D, T = x.shape
    N = embeddings.shape[0]
    xv = x.reshape(B, T, D)

    q_flat, parts = pl.pallas_call(
        _vq_kernel,
        grid=(B,),
        in_specs=[
            pl.BlockSpec((1, D, T), lambda b: (b, 0, 0)),
            pl.BlockSpec((1, T, D), lambda b: (b, 0, 0)),
            pl.BlockSpec((N, D), lambda b: (0, 0)),
        ],
        out_specs=[
            pl.BlockSpec((1, T, D), lambda b: (b, 0, 0)),
            pl.BlockSpec((1, 1, 128), lambda b: (b, 0, 0)),
        ],
        out_shape=[
            jax.ShapeDtypeStruct((B, T, D), jnp.float32),
            jax.ShapeDtypeStruct((B, 1, 128), jnp.float32),
        ],
    )(x, xv, embeddings)

    quantized = q_flat.reshape(B, D, T)
    total = jnp.sum(parts[:, 0, 0])
    loss = 1.25 * total / (B * D * T)
    return (quantized, loss)


# speedup vs baseline: 2.9970x; 2.4506x over previous
import functools
import jax
import jax.numpy as jnp
from jax import lax
from jax.experimental import pallas as pl
from jax.experimental.pallas import tpu as pltpu
from jax.experimental.pallas import tpu_sc as plsc

NC = 2
NS = 16
NW = NC * NS


def _tree_sum_rows(a):
    n = a.shape[0]
    while n > 1:
        n //= 2
        a = a[:n] + a[n:]
    return a


def _tree_sum_cols(a):
    n = a.shape[1]
    while n > 1:
        n //= 2
        a = a[:, :n] + a[:, n:]
    return a


def _argmin_kernel(x_ref, emb_ref, idx_ref):
    xb = x_ref[0]
    e = emb_ref[...]

    xsq = _tree_sum_rows(xb * xb)
    esq = _tree_sum_cols(e * e)

    mm = lax.dot_general(e, xb, (((1,), (0,)), ((), ())),
                         preferred_element_type=jnp.float32)
    dist = (xsq + esq) - 2.0 * mm

    n, t = dist.shape
    iota_c = lax.broadcasted_iota(jnp.int32, (n, t), 0)
    mn = jnp.min(dist, axis=0, keepdims=True)
    idx_ref[0, 0] = jnp.min(jnp.where(dist == mn, iota_c, n), axis=0)


def _sc_gather_loss(emb_hbm, idx_hbm, xv_hbm, q_hbm, part_hbm,
                    table_v, idx_v, rows_v, xv_v, acc_v, *, tok, dim, ch):
    per_w = tok // NW
    nch = per_w // ch
    wid = lax.axis_index("s") * NC + lax.axis_index("c")

    pltpu.sync_copy(emb_hbm, table_v)

    def chunk_body(ci, acc):
        base = wid * per_w + ci * ch
        pltpu.sync_copy(idx_hbm.at[pl.ds(base, ch)], idx_v)
        pltpu.sync_copy(xv_hbm.at[pl.ds(base * dim, ch * dim)], xv_v)

        def grp_body(p, a):
            iv = idx_v[pl.ds(p * 16, 16)]
            for k in range(16):
                r = p * 16 + k
                off = iv[k] * dim
                q0 = table_v[pl.ds(off, 16)]
                q1 = table_v[pl.ds(off + 16, 16)]
                rows_v[pl.ds(r * dim, 16)] = q0
                rows_v[pl.ds(r * dim + 16, 16)] = q1
                d0 = q0 - xv_v[pl.ds(r * dim, 16)]
                d1 = q1 - xv_v[pl.ds(r * dim + 16, 16)]
                a = a + (d0 * d0 + d1 * d1)
            return a

        acc = lax.fori_loop(0, ch // 16, grp_body, acc)
        pltpu.sync_copy(rows_v, q_hbm.at[pl.ds(base * dim, ch * dim)])
        return acc

    acc = lax.fori_loop(0, nch, chunk_body, jnp.zeros((16,), jnp.float32))
    acc_v[...] = acc
    pltpu.sync_copy(acc_v, part_hbm.at[pl.ds(wid * 16, 16)])


def kernel(x, embeddings):
    B, D, T = x.shape
    N = embeddings.shape[0]
    tok = B * T
    ch = 512

    idx3 = pl.pallas_call(
        _argmin_kernel,
        grid=(B,),
        in_specs=[
            pl.BlockSpec((1, D, T), lambda b: (b, 0, 0)),
            pl.BlockSpec((N, D), lambda b: (0, 0)),
        ],
        out_specs=pl.BlockSpec((1, 1, T), lambda b: (b, 0, 0)),
        out_shape=jax.ShapeDtypeStruct((B, 1, T), jnp.int32),
    )(x, embeddings)

    idx_flat = idx3.reshape(tok)
    xv = x.reshape(tok * D)

    sc_fn = functools.partial(_sc_gather_loss, tok=tok, dim=D, ch=ch)
    q_flat, parts = pl.kernel(
        sc_fn,
        out_type=[
            jax.ShapeDtypeStruct((tok * D,), jnp.float32),
            jax.ShapeDtypeStruct((NW * 16,), jnp.float32),
        ],
        mesh=plsc.VectorSubcoreMesh(core_axis_name="c", subcore_axis_name="s"),
        scratch_types=[
            pltpu.VMEM((N * D,), jnp.float32),
            pltpu.VMEM((ch,), jnp.int32),
            pltpu.VMEM((ch * D,), jnp.float32),
            pltpu.VMEM((ch * D,), jnp.float32),
            pltpu.VMEM((16,), jnp.float32),
        ],
    )(embeddings.reshape(N * D), idx_flat, xv)

    quantized = q_flat.reshape(B, D, T)
    loss = 1.25 * jnp.sum(parts) / (B * D * T)
    return (quantized, loss)


# speedup vs baseline: 3.2117x; 1.0716x over previous
import functools
import jax
import jax.numpy as jnp
from jax import lax
from jax.experimental import pallas as pl
from jax.experimental.pallas import tpu as pltpu
from jax.experimental.pallas import tpu_sc as plsc

NC = 2
NS = 16
NW = NC * NS


def _tree_sum_rows(a):
    n = a.shape[0]
    while n > 1:
        n //= 2
        a = a[:n] + a[n:]
    return a


def _tree_sum_cols(a):
    n = a.shape[1]
    while n > 1:
        n //= 2
        a = a[:, :n] + a[:, n:]
    return a


def _argmin_kernel(x_ref, emb_ref, idx_ref):
    xb = x_ref[0]
    e = emb_ref[...]

    xsq = _tree_sum_rows(xb * xb)
    esq = _tree_sum_cols(e * e)

    mm = lax.dot_general(e, xb, (((1,), (0,)), ((), ())),
                         preferred_element_type=jnp.float32)
    dist = (xsq + esq) - 2.0 * mm

    n, t = dist.shape
    iota_c = lax.broadcasted_iota(jnp.int32, (n, t), 0)
    mn = jnp.min(dist, axis=0, keepdims=True)
    idx_ref[0, 0] = jnp.min(jnp.where(dist == mn, iota_c, n), axis=0)


def _sc_gather_loss(emb_hbm, idx_hbm, x_hbm, q_hbm, part_hbm,
                    table_v, idx_v, rows_v, xv_v, acc_v, *, tok, dim, t_len, ch):
    per_w = tok // NW
    nch = per_w // ch
    rows_per_chunk = ch * dim // t_len
    wid = lax.axis_index("s") * NC + lax.axis_index("c")

    pltpu.sync_copy(emb_hbm, table_v)

    def chunk_body(ci, acc):
        base = wid * per_w + ci * ch
        b = base // t_len
        t0 = pl.multiple_of(base % t_len, ch)
        d0 = pl.multiple_of((base % t_len) * dim // t_len, rows_per_chunk)
        pltpu.sync_copy(idx_hbm.at[b, 0, pl.ds(t0, ch)], idx_v)
        pltpu.sync_copy(x_hbm.at[b, pl.ds(d0, rows_per_chunk)], xv_v)

        def grp_body(p, a):
            iv = idx_v[pl.ds(p * 16, 16)]
            row = p // 2
            for k in range(16):
                col = ((p % 2) * 16 + k) * dim
                i = iv[k]
                q0 = table_v[i, pl.ds(0, 16)]
                q1 = table_v[i, pl.ds(16, 16)]
                rows_v[row, pl.ds(col, 16)] = q0
                rows_v[row, pl.ds(col + 16, 16)] = q1
                d0v = q0 - xv_v[row, pl.ds(col, 16)]
                d1v = q1 - xv_v[row, pl.ds(col + 16, 16)]
                a = a + (d0v * d0v + d1v * d1v)
            return a

        acc = lax.fori_loop(0, ch // 16, grp_body, acc)
        pltpu.sync_copy(rows_v, q_hbm.at[b, pl.ds(d0, rows_per_chunk)])
        return acc

    acc = lax.fori_loop(0, nch, chunk_body, jnp.zeros((16,), jnp.float32))
    acc_v[...] = acc
    pltpu.sync_copy(acc_v, part_hbm.at[pl.ds(wid * 16, 16)])


def kernel(x, embeddings):
    B, D, T = x.shape
    N = embeddings.shape[0]
    tok = B * T
    ch = 512

    idx3 = pl.pallas_call(
        _argmin_kernel,
        grid=(B,),
        in_specs=[
            pl.BlockSpec((1, D, T), lambda b: (b, 0, 0)),
            pl.BlockSpec((N, D), lambda b: (0, 0)),
        ],
        out_specs=pl.BlockSpec((1, 1, T), lambda b: (b, 0, 0)),
        out_shape=jax.ShapeDtypeStruct((B, 1, T), jnp.int32),
    )(x, embeddings)

    rows_per_chunk = ch * D // T
    sc_fn = functools.partial(_sc_gather_loss, tok=tok, dim=D, t_len=T, ch=ch)
    quantized, parts = pl.kernel(
        sc_fn,
        out_type=[
            jax.ShapeDtypeStruct((B, D, T), jnp.float32),
            jax.ShapeDtypeStruct((NW * 16,), jnp.float32),
        ],
        mesh=plsc.VectorSubcoreMesh(core_axis_name="c", subcore_axis_name="s"),
        scratch_types=[
            pltpu.VMEM((N, D), jnp.float32),
            pltpu.VMEM((ch,), jnp.int32),
            pltpu.VMEM((rows_per_chunk, T), jnp.float32),
            pltpu.VMEM((rows_per_chunk, T), jnp.float32),
            pltpu.VMEM((16,), jnp.float32),
        ],
    )(embeddings, idx3, x)

    loss = 1.25 * jnp.sum(parts) / (B * D * T)
    return (quantized, loss)


# speedup vs baseline: 3.5629x; 1.1093x over previous
import functools
import jax
import jax.numpy as jnp
from jax import lax
from jax.experimental import pallas as pl
from jax.experimental.pallas import tpu as pltpu
from jax.experimental.pallas import tpu_sc as plsc

NC = 2
NS = 16
NW = NC * NS


def _tree_sum_rows(a):
    n = a.shape[0]
    while n > 1:
        n //= 2
        a = a[:n] + a[n:]
    return a


def _tree_sum_cols(a):
    n = a.shape[1]
    while n > 1:
        n //= 2
        a = a[:, :n] + a[:, n:]
    return a


def _argmin_kernel(x_ref, emb_ref, idx_ref):
    xb = x_ref[0]
    e = emb_ref[...]

    xsq = _tree_sum_rows(xb * xb)
    esq = _tree_sum_cols(e * e)

    mm = lax.dot_general(e, xb, (((1,), (0,)), ((), ())),
                         preferred_element_type=jnp.float32)
    dist = (xsq + esq) - 2.0 * mm

    n, t = dist.shape
    val = dist
    idx = lax.broadcasted_iota(jnp.int32, (n, t), 0)
    while val.shape[0] > 1:
        h = val.shape[0] // 2
        take_b = val[h:] < val[:h]
        val = jnp.where(take_b, val[h:], val[:h])
        idx = jnp.where(take_b, idx[h:], idx[:h])
    idx_ref[0, 0] = idx[0]


def _sc_gather_loss(emb_hbm, idx_hbm, x_hbm, q_hbm, part_hbm,
                    table_v, idx_v, rows_v, xv_v, acc_v, *, tok, dim, t_len, ch):
    per_w = tok // NW
    nch = per_w // ch
    rows_per_chunk = ch * dim // t_len
    wid = lax.axis_index("s") * NC + lax.axis_index("c")

    pltpu.sync_copy(emb_hbm, table_v)

    def chunk_body(ci, acc):
        base = wid * per_w + ci * ch
        b = base // t_len
        t0 = pl.multiple_of(base % t_len, ch)
        d0 = pl.multiple_of((base % t_len) * dim // t_len, rows_per_chunk)
        pltpu.sync_copy(idx_hbm.at[b, 0, pl.ds(t0, ch)], idx_v)
        pltpu.sync_copy(x_hbm.at[b, pl.ds(d0, rows_per_chunk)], xv_v)

        def grp_body(p, a):
            iv = idx_v[pl.ds(p * 16, 16)]
            row = p // 2
            for k in range(16):
                col = ((p % 2) * 16 + k) * dim
                i = iv[k]
                q0 = table_v[i, pl.ds(0, 16)]
                q1 = table_v[i, pl.ds(16, 16)]
                rows_v[row, pl.ds(col, 16)] = q0
                rows_v[row, pl.ds(col + 16, 16)] = q1
                d0v = q0 - xv_v[row, pl.ds(col, 16)]
                d1v = q1 - xv_v[row, pl.ds(col + 16, 16)]
                a = a + (d0v * d0v + d1v * d1v)
            return a

        acc = lax.fori_loop(0, ch // 16, grp_body, acc)
        pltpu.sync_copy(rows_v, q_hbm.at[b, pl.ds(d0, rows_per_chunk)])
        return acc

    acc = lax.fori_loop(0, nch, chunk_body, jnp.zeros((16,), jnp.float32))
    acc_v[...] = acc
    pltpu.sync_copy(acc_v, part_hbm.at[pl.ds(wid * 16, 16)])


def kernel(x, embeddings):
    B, D, T = x.shape
    N = embeddings.shape[0]
    tok = B * T
    ch = 512

    idx3 = pl.pallas_call(
        _argmin_kernel,
        grid=(B,),
        in_specs=[
            pl.BlockSpec((1, D, T), lambda b: (b, 0, 0)),
            pl.BlockSpec((N, D), lambda b: (0, 0)),
        ],
        out_specs=pl.BlockSpec((1, 1, T), lambda b: (b, 0, 0)),
        out_shape=jax.ShapeDtypeStruct((B, 1, T), jnp.int32),
    )(x, embeddings)

    rows_per_chunk = ch * D // T
    sc_fn = functools.partial(_sc_gather_loss, tok=tok, dim=D, t_len=T, ch=ch)
    quantized, parts = pl.kernel(
        sc_fn,
        out_type=[
            jax.ShapeDtypeStruct((B, D, T), jnp.float32),
            jax.ShapeDtypeStruct((NW * 16,), jnp.float32),
        ],
        mesh=plsc.VectorSubcoreMesh(core_axis_name="c", subcore_axis_name="s"),
        scratch_types=[
            pltpu.VMEM((N, D), jnp.float32),
            pltpu.VMEM((ch,), jnp.int32),
            pltpu.VMEM((rows_per_chunk, T), jnp.float32),
            pltpu.VMEM((rows_per_chunk, T), jnp.float32),
            pltpu.VMEM((16,), jnp.float32),
        ],
    )(embeddings, idx3, x)

    loss = 1.25 * jnp.sum(parts) / (B * D * T)
    return (quantized, loss)


# speedup vs baseline: 3.8110x; 1.0696x over previous
import functools
import jax
import jax.numpy as jnp
from jax import lax
from jax.experimental import pallas as pl
from jax.experimental.pallas import tpu as pltpu
from jax.experimental.pallas import tpu_sc as plsc

NC = 2
NS = 16
NW = NC * NS


def _tree_sum_rows(a):
    n = a.shape[0]
    while n > 1:
        n //= 2
        a = a[:n] + a[n:]
    return a


def _tree_sum_cols(a):
    n = a.shape[1]
    while n > 1:
        n //= 2
        a = a[:, :n] + a[:, n:]
    return a


def _argmin_kernel(x_ref, emb_ref, idx_ref):
    xb = x_ref[0]
    e = emb_ref[...]

    xsq = _tree_sum_rows(xb * xb)
    esq = _tree_sum_cols(e * e)

    mm2 = lax.dot_general(e + e, xb, (((1,), (0,)), ((), ())),
                          preferred_element_type=jnp.float32)

    n = mm2.shape[0]
    t = mm2.shape[1]
    blk = 8
    iota_b = lax.broadcasted_iota(jnp.int32, (blk, t), 0)

    def dist_blk(k):
        return (xsq + esq[k * blk:(k + 1) * blk]) - mm2[k * blk:(k + 1) * blk]

    val = dist_blk(0)
    idx = iota_b
    for k in range(1, n // blk):
        vb = dist_blk(k)
        take_b = vb < val
        val = jnp.where(take_b, vb, val)
        idx = jnp.where(take_b, iota_b + (k * blk), idx)

    while val.shape[0] > 1:
        h = val.shape[0] // 2
        va, vb = val[:h], val[h:]
        ia, ib = idx[:h], idx[h:]
        take_b = (vb < va) | ((vb == va) & (ib < ia))
        val = jnp.where(take_b, vb, va)
        idx = jnp.where(take_b, ib, ia)
    idx_ref[0, 0] = idx[0]


def _sc_gather_loss(emb_hbm, idx_hbm, x_hbm, q_hbm, part_hbm,
                    table_v, idx_v, rows_v, xv_v, acc_v, *, tok, dim, t_len, ch):
    per_w = tok // NW
    nch = per_w // ch
    rows_per_chunk = ch * dim // t_len
    wid = lax.axis_index("s") * NC + lax.axis_index("c")

    pltpu.sync_copy(emb_hbm, table_v)

    def chunk_body(ci, acc):
        base = wid * per_w + ci * ch
        b = base // t_len
        t0 = pl.multiple_of(base % t_len, ch)
        d0 = pl.multiple_of((base % t_len) * dim // t_len, rows_per_chunk)
        pltpu.sync_copy(idx_hbm.at[b, 0, pl.ds(t0, ch)], idx_v)
        pltpu.sync_copy(x_hbm.at[b, pl.ds(d0, rows_per_chunk)], xv_v)

        def grp_body(p, a):
            iv = idx_v[pl.ds(p * 16, 16)]
            row = p // 2
            for k in range(16):
                col = ((p % 2) * 16 + k) * dim
                i = iv[k]
                q0 = table_v[i, pl.ds(0, 16)]
                q1 = table_v[i, pl.ds(16, 16)]
                rows_v[row, pl.ds(col, 16)] = q0
                rows_v[row, pl.ds(col + 16, 16)] = q1
                d0v = q0 - xv_v[row, pl.ds(col, 16)]
                d1v = q1 - xv_v[row, pl.ds(col + 16, 16)]
                a = a + (d0v * d0v + d1v * d1v)
            return a

        acc = lax.fori_loop(0, ch // 16, grp_body, acc)
        pltpu.sync_copy(rows_v, q_hbm.at[b, pl.ds(d0, rows_per_chunk)])
        return acc

    acc = lax.fori_loop(0, nch, chunk_body, jnp.zeros((16,), jnp.float32))
    acc_v[...] = acc
    pltpu.sync_copy(acc_v, part_hbm.at[pl.ds(wid * 16, 16)])


def kernel(x, embeddings):
    B, D, T = x.shape
    N = embeddings.shape[0]
    tok = B * T
    ch = 512

    idx3 = pl.pallas_call(
        _argmin_kernel,
        grid=(B,),
        in_specs=[
            pl.BlockSpec((1, D, T), lambda b: (b, 0, 0)),
            pl.BlockSpec((N, D), lambda b: (0, 0)),
        ],
        out_specs=pl.BlockSpec((1, 1, T), lambda b: (b, 0, 0)),
        out_shape=jax.ShapeDtypeStruct((B, 1, T), jnp.int32),
    )(x, embeddings)

    rows_per_chunk = ch * D // T
    sc_fn = functools.partial(_sc_gather_loss, tok=tok, dim=D, t_len=T, ch=ch)
    quantized, parts = pl.kernel(
        sc_fn,
        out_type=[
            jax.ShapeDtypeStruct((B, D, T), jnp.float32),
            jax.ShapeDtypeStruct((NW * 16,), jnp.float32),
        ],
        mesh=plsc.VectorSubcoreMesh(core_axis_name="c", subcore_axis_name="s"),
        scratch_types=[
            pltpu.VMEM((N, D), jnp.float32),
            pltpu.VMEM((ch,), jnp.int32),
            pltpu.VMEM((rows_per_chunk, T), jnp.float32),
            pltpu.VMEM((rows_per_chunk, T), jnp.float32),
            pltpu.VMEM((16,), jnp.float32),
        ],
    )(embeddings, idx3, x)

    loss = 1.25 * jnp.sum(parts) / (B * D * T)
    return (quantized, loss)


# speedup vs baseline: 4.3260x; 1.1351x over previous
import functools
import jax
import jax.numpy as jnp
from jax import lax
from jax.experimental import pallas as pl
from jax.experimental.pallas import tpu as pltpu
from jax.experimental.pallas import tpu_sc as plsc

NC = 2
NS = 16
NW = NC * NS


def _tree_sum_rows(a):
    n = a.shape[0]
    while n > 1:
        n //= 2
        a = a[:n] + a[n:]
    return a


def _tree_sum_cols(a):
    n = a.shape[1]
    while n > 1:
        n //= 2
        a = a[:, :n] + a[:, n:]
    return a


def _argmin_kernel(x_ref, emb_ref, idx_ref):
    xb = x_ref[0]
    e = emb_ref[...]

    xsq = _tree_sum_rows(xb * xb)
    esq = _tree_sum_cols(e * e)

    mm2 = lax.dot_general(e + e, xb, (((1,), (0,)), ((), ())),
                          preferred_element_type=jnp.float32)

    n = mm2.shape[0]
    t = mm2.shape[1]
    blk = 8
    iota_b = lax.broadcasted_iota(jnp.int32, (blk, t), 0)

    def dist_blk(k):
        return (xsq + esq[k * blk:(k + 1) * blk]) - mm2[k * blk:(k + 1) * blk]

    val = dist_blk(0)
    idx = iota_b
    for k in range(1, n // blk):
        vb = dist_blk(k)
        take_b = vb < val
        val = jnp.where(take_b, vb, val)
        idx = jnp.where(take_b, iota_b + (k * blk), idx)

    while val.shape[0] > 1:
        h = val.shape[0] // 2
        va, vb = val[:h], val[h:]
        ia, ib = idx[:h], idx[h:]
        take_b = (vb < va) | ((vb == va) & (ib < ia))
        val = jnp.where(take_b, vb, va)
        idx = jnp.where(take_b, ib, ia)
    idx_ref[0, 0] = idx[0]


def _sc_gather_loss(emb_hbm, idx_hbm, x_hbm, q_hbm, part_hbm,
                    table_v, idx_v, rows_v, xv_v, acc_v,
                    si0, si1, sx0, sx1, sq0, sq1, *, tok, dim, t_len, ch):
    per_w = tok // NW
    nch = per_w // ch
    rpc = ch * dim // t_len
    wid = lax.axis_index("s") * NC + lax.axis_index("c")

    pltpu.sync_copy(emb_hbm, table_v)

    def addr(ci):
        base = wid * per_w + ci * ch
        b = base // t_len
        t0 = pl.multiple_of(base % t_len, ch)
        d0 = pl.multiple_of((base % t_len) * dim // t_len, rpc)
        return b, t0, d0

    sems_i = [si0, si1]
    sems_x = [sx0, sx1]
    sems_q = [sq0, sq1]

    def start_in(ci, s):
        b, t0, d0 = addr(ci)
        ci_ = pltpu.async_copy(idx_hbm.at[b, 0, pl.ds(t0, ch)],
                               idx_v.at[s], sems_i[s])
        cx_ = pltpu.async_copy(x_hbm.at[b, pl.ds(d0, rpc)],
                               xv_v.at[s], sems_x[s])
        return ci_, cx_

    acc = jnp.zeros((16,), jnp.float32)
    pend_in = start_in(0, 0)
    pend_out = [None, None]
    for ci in range(nch):
        s = ci % 2
        pend_in[0].wait()
        pend_in[1].wait()
        if ci + 1 < nch:
            pend_in = start_in(ci + 1, 1 - s)
        if pend_out[s] is not None:
            pend_out[s].wait()

        def grp_body(p, a, s=s):
            iv = idx_v[s, pl.ds(p * 16, 16)]
            row = p // 2
            for k in range(16):
                col = ((p % 2) * 16 + k) * dim
                off = iv[k] * dim
                q0 = table_v[pl.ds(off, 16)]
                q1 = table_v[pl.ds(off + 16, 16)]
                rows_v[s, row, pl.ds(col, 16)] = q0
                rows_v[s, row, pl.ds(col + 16, 16)] = q1
                d0v = q0 - xv_v[s, row, pl.ds(col, 16)]
                d1v = q1 - xv_v[s, row, pl.ds(col + 16, 16)]
                a = a + (d0v * d0v + d1v * d1v)
            return a

        acc = lax.fori_loop(0, ch // 16, grp_body, acc)
        b, t0, d0 = addr(ci)
        pend_out[s] = pltpu.async_copy(rows_v.at[s],
                                       q_hbm.at[b, pl.ds(d0, rpc)], sems_q[s])
    pend_out[0].wait()
    pend_out[1].wait()
    acc_v[...] = acc
    pltpu.sync_copy(acc_v, part_hbm.at[pl.ds(wid * 16, 16)])


def kernel(x, embeddings):
    B, D, T = x.shape
    N = embeddings.shape[0]
    tok = B * T
    ch = 512

    idx3 = pl.pallas_call(
        _argmin_kernel,
        grid=(B,),
        in_specs=[
            pl.BlockSpec((1, D, T), lambda b: (b, 0, 0)),
            pl.BlockSpec((N, D), lambda b: (0, 0)),
        ],
        out_specs=pl.BlockSpec((1, 1, T), lambda b: (b, 0, 0)),
        out_shape=jax.ShapeDtypeStruct((B, 1, T), jnp.int32),
    )(x, embeddings)

    rows_per_chunk = ch * D // T
    sc_fn = functools.partial(_sc_gather_loss, tok=tok, dim=D, t_len=T, ch=ch)
    quantized, parts = pl.kernel(
        sc_fn,
        out_type=[
            jax.ShapeDtypeStruct((B, D, T), jnp.float32),
            jax.ShapeDtypeStruct((NW * 16,), jnp.float32),
        ],
        mesh=plsc.VectorSubcoreMesh(core_axis_name="c", subcore_axis_name="s"),
        scratch_types=[
            pltpu.VMEM((N * D,), jnp.float32),
            pltpu.VMEM((2, ch), jnp.int32),
            pltpu.VMEM((2, rows_per_chunk, T), jnp.float32),
            pltpu.VMEM((2, rows_per_chunk, T), jnp.float32),
            pltpu.VMEM((16,), jnp.float32),
            pltpu.SemaphoreType.DMA,
            pltpu.SemaphoreType.DMA,
            pltpu.SemaphoreType.DMA,
            pltpu.SemaphoreType.DMA,
            pltpu.SemaphoreType.DMA,
            pltpu.SemaphoreType.DMA,
        ],
    )(embeddings.reshape(N * D), idx3, x)

    loss = 1.25 * jnp.sum(parts) / (B * D * T)
    return (quantized, loss)


# speedup vs baseline: 6.2410x; 1.4427x over previous
import functools
import jax
import jax.numpy as jnp
from jax import lax
from jax.experimental import pallas as pl
from jax.experimental.pallas import tpu as pltpu
from jax.experimental.pallas import tpu_sc as plsc

NC = 2
NS = 16
NW = NC * NS


def _tree_sum_rows(a):
    n = a.shape[0]
    while n > 1:
        n //= 2
        a = a[:n] + a[n:]
    return a


def _tree_sum_cols(a):
    n = a.shape[1]
    while n > 1:
        n //= 2
        a = a[:, :n] + a[:, n:]
    return a


def _argmin_kernel(x_ref, emb_ref, idx_ref):
    e = emb_ref[...]
    for j in range(x_ref.shape[1]):
        _argmin_one(x_ref[0, j], e, idx_ref, j)


def _argmin_one(xb, e, idx_ref, j):

    xsq = _tree_sum_rows(xb * xb)
    esq = _tree_sum_cols(e * e)

    mm2 = lax.dot_general(e + e, xb, (((1,), (0,)), ((), ())),
                          preferred_element_type=jnp.float32)

    n = mm2.shape[0]
    t = mm2.shape[1]
    blk = 8
    iota_b = lax.broadcasted_iota(jnp.int32, (blk, t), 0)

    def dist_blk(k):
        return (xsq + esq[k * blk:(k + 1) * blk]) - mm2[k * blk:(k + 1) * blk]

    val = dist_blk(0)
    idx = iota_b
    for k in range(1, n // blk):
        vb = dist_blk(k)
        take_b = vb < val
        val = jnp.where(take_b, vb, val)
        idx = jnp.where(take_b, iota_b + (k * blk), idx)

    while val.shape[0] > 1:
        h = val.shape[0] // 2
        va, vb = val[:h], val[h:]
        ia, ib = idx[:h], idx[h:]
        take_b = (vb < va) | ((vb == va) & (ib < ia))
        val = jnp.where(take_b, vb, va)
        idx = jnp.where(take_b, ib, ia)
    idx_ref[0, j] = idx[0]


def _sc_gather_loss(emb_hbm, idx_hbm, x_hbm, q_hbm, part_hbm,
                    table_v, idx_v, rows_v, xv_v, acc_v,
                    si0, si1, sx0, sx1, sq0, sq1, *, tok, dim, t_len, ch):
    per_w = tok // NW
    nch = per_w // ch
    rpc = ch * dim // t_len
    wid = lax.axis_index("s") * NC + lax.axis_index("c")

    pltpu.sync_copy(emb_hbm, table_v)

    def addr(ci):
        base = wid * per_w + ci * ch
        b = base // t_len
        t0 = pl.multiple_of(base % t_len, ch)
        d0 = pl.multiple_of((base % t_len) * dim // t_len, rpc)
        return b, t0, d0

    sems_i = [si0, si1]
    sems_x = [sx0, sx1]
    sems_q = [sq0, sq1]

    def start_in(ci, s):
        b, t0, d0 = addr(ci)
        ci_ = pltpu.async_copy(idx_hbm.at[b // 8, b % 8, pl.ds(t0, ch)],
                               idx_v.at[s], sems_i[s])
        cx_ = pltpu.async_copy(x_hbm.at[b, pl.ds(d0, rpc)],
                               xv_v.at[s], sems_x[s])
        return ci_, cx_

    acc = jnp.zeros((16,), jnp.float32)
    pend_in = start_in(0, 0)
    pend_out = [None, None]
    for ci in range(nch):
        s = ci % 2
        pend_in[0].wait()
        pend_in[1].wait()
        if ci + 1 < nch:
            pend_in = start_in(ci + 1, 1 - s)
        if pend_out[s] is not None:
            pend_out[s].wait()

        def grp_body(p, a, s=s):
            iv = idx_v[s, pl.ds(p * 16, 16)]
            row = p // 2
            for k in range(16):
                col = ((p % 2) * 16 + k) * dim
                off = iv[k] * dim
                q0 = table_v[pl.ds(off, 16)]
                q1 = table_v[pl.ds(off + 16, 16)]
                rows_v[s, row, pl.ds(col, 16)] = q0
                rows_v[s, row, pl.ds(col + 16, 16)] = q1
                d0v = q0 - xv_v[s, row, pl.ds(col, 16)]
                d1v = q1 - xv_v[s, row, pl.ds(col + 16, 16)]
                a = a + (d0v * d0v + d1v * d1v)
            return a

        acc = lax.fori_loop(0, ch // 16, grp_body, acc)
        b, t0, d0 = addr(ci)
        pend_out[s] = pltpu.async_copy(rows_v.at[s],
                                       q_hbm.at[b, pl.ds(d0, rpc)], sems_q[s])
    pend_out[0].wait()
    pend_out[1].wait()
    acc_v[...] = acc
    pltpu.sync_copy(acc_v, part_hbm.at[pl.ds(wid * 16, 16)])


def kernel(x, embeddings):
    B, D, T = x.shape
    N = embeddings.shape[0]
    tok = B * T
    ch = 512

    bb = 8
    idx3 = pl.pallas_call(
        _argmin_kernel,
        grid=(B // bb,),
        in_specs=[
            pl.BlockSpec((1, bb, D, T), lambda b: (b, 0, 0, 0)),
            pl.BlockSpec((N, D), lambda b: (0, 0)),
        ],
        out_specs=pl.BlockSpec((1, bb, T), lambda b: (b, 0, 0)),
        out_shape=jax.ShapeDtypeStruct((B // bb, bb, T), jnp.int32),
    )(x.reshape(B // bb, bb, D, T), embeddings)

    rows_per_chunk = ch * D // T
    sc_fn = functools.partial(_sc_gather_loss, tok=tok, dim=D, t_len=T, ch=ch)
    quantized, parts = pl.kernel(
        sc_fn,
        out_type=[
            jax.ShapeDtypeStruct((B, D, T), jnp.float32),
            jax.ShapeDtypeStruct((NW * 16,), jnp.float32),
        ],
        mesh=plsc.VectorSubcoreMesh(core_axis_name="c", subcore_axis_name="s"),
        scratch_types=[
            pltpu.VMEM((N * D,), jnp.float32),
            pltpu.VMEM((2, ch), jnp.int32),
            pltpu.VMEM((2, rows_per_chunk, T), jnp.float32),
            pltpu.VMEM((2, rows_per_chunk, T), jnp.float32),
            pltpu.VMEM((16,), jnp.float32),
            pltpu.SemaphoreType.DMA,
            pltpu.SemaphoreType.DMA,
            pltpu.SemaphoreType.DMA,
            pltpu.SemaphoreType.DMA,
            pltpu.SemaphoreType.DMA,
            pltpu.SemaphoreType.DMA,
        ],
    )(embeddings.reshape(N * D), idx3, x)

    loss = 1.25 * jnp.sum(parts) / (B * D * T)
    return (quantized, loss)
